# baseline (speedup 1.0000x reference)
import functools

import jax
import jax.numpy as jnp
from jax import lax
from jax.experimental import pallas as pl
from jax.experimental.pallas import tpu as pltpu
from jax.experimental.pallas import tpu_sc as plsc

C, H, W = 768, 128, 128
R = 4096
N_PTS = 32
NC, NS, L = 2, 16, 16
NW = NC * NS
LINES_PER_W = R // NW
NG = C // 32
OPAD = N_PTS + 1

_GDN = lax.GatherDimensionNumbers(
    offset_dims=(), collapsed_slice_dims=(0,), start_index_map=(0,))


def _splat(vec, p):
    idx = jnp.full((L, 1), p, dtype=jnp.int32)
    return lax.gather(vec, idx, _GDN, (1,),
                      mode=lax.GatherScatterMode.PROMISE_IN_BOUNDS)


def _rne_bf16_bits(x):
    u = lax.bitcast_convert_type(x, jnp.int32)
    return (u + 0x7FFF + ((u >> 16) & 1)) >> 16


def _transpose_body(x_ref, o_ref):
    x = x_ref[...].reshape(C // 2, 2, -1)
    we = _rne_bf16_bits(x[:, 0, :]) & 0xFFFF
    wo = _rne_bf16_bits(x[:, 1, :]) << 16
    packed = jnp.transpose(we | wo, (1, 0))
    o_ref[...] = lax.bitcast_convert_type(packed, jnp.float32)


def _make_table(feature):
    return pl.pallas_call(
        _transpose_body,
        grid=(16,),
        in_specs=[pl.BlockSpec((C, H * W // 16), lambda i: (0, i))],
        out_specs=pl.BlockSpec((H * W // 16, C // 2), lambda i: (i, 0)),
        out_shape=jax.ShapeDtypeStruct((H * W, C // 2), jnp.float32),
    )(feature.reshape(C, H * W))


def _geometry(loi_v, lams, line, half):
    ctrl = loi_v[pl.ds(line * 6, L)]
    x0 = _splat(ctrl, 0)
    y0 = _splat(ctrl, 1)
    x1 = _splat(ctrl, 2)
    y1 = _splat(ctrl, 3)
    x2 = _splat(ctrl, 4)
    y2 = _splat(ctrl, 5)
    l0, l1, l2 = lams[half]
    px = l0 * x0 + l1 * x1 + l2 * x2 - 0.5
    py = l0 * y0 + l1 * y1 + l2 * y2 - 0.5
    tx = px.astype(jnp.int32).astype(jnp.float32)
    fx = jnp.where(tx > px, tx - 1.0, tx)
    ty = py.astype(jnp.int32).astype(jnp.float32)
    fy = jnp.where(ty > py, ty - 1.0, ty)
    px0 = jnp.minimum(jnp.maximum(fx, 0.0), W - 1.0)
    py0 = jnp.minimum(jnp.maximum(fy, 0.0), H - 1.0)
    px1 = jnp.minimum(px0 + 1.0, W - 1.0)
    py1 = jnp.minimum(py0 + 1.0, H - 1.0)
    wxa = px1 - px
    wxb = px - px0
    wya = py1 - py
    wyb = py - py0
    ix0 = px0.astype(jnp.int32)
    ix1 = px1.astype(jnp.int32)
    iy0 = py0.astype(jnp.int32) * W
    iy1 = py1.astype(jnp.int32) * W
    idx = (iy0 + ix0, iy1 + ix0, iy0 + ix1, iy1 + ix1)
    wts = (wya * wxa, wyb * wxa, wya * wxb, wyb * wxb)
    return idx, wts


def _loi_body(table, loi, lam, out, lam_v, loi_v,
              ia0, ib0, ic0, id0, ia1, ib1, ic1, id1,
              ia2, ib2, ic2, id2,
              ba0, bb0, bc0, bd0, ba1, bb1, bc1, bd1,
              ba2, bb2, bc2, bd2,
              out_t, cbuf, gsem0, gsem1, gsem2, osem):
    wid = lax.axis_index("s") * NC + lax.axis_index("c")
    base = wid * LINES_PER_W
    pltpu.sync_copy(lam, lam_v)
    pltpu.sync_copy(loi.at[pl.ds(base * 6, LINES_PER_W * 6)],
                    loi_v.at[pl.ds(0, LINES_PER_W * 6)])
    iota = lax.iota(jnp.int32, L)
    iota66 = iota * (2 * OPAD)
    lams = tuple(
        (lam_v[0, pl.ds(L * h, L)], lam_v[1, pl.ds(L * h, L)],
         lam_v[2, pl.ds(L * h, L)])
        for h in range(2))
    idx_sets = ((ia0, ib0, ic0, id0), (ia1, ib1, ic1, id1),
                (ia2, ib2, ic2, id2))
    buf_sets = ((ba0, bb0, bc0, bd0), (ba1, bb1, bc1, bd1),
                (ba2, bb2, bc2, bd2))
    gsems = (gsem0, gsem1, gsem2)

    def fire(line, half, t):
        idx, _ = _geometry(loi_v, lams, line, half)
        for k in range(4):
            idx_sets[t][k][...] = idx[k]
        for k in range(4):
            pltpu.async_copy(table.at[idx_sets[t][k]], buf_sets[t][k],
                             gsems[t])

    def drain_gathers(t):
        for k in range(4):
            pltpu.make_async_copy(table.at[idx_sets[t][k]], buf_sets[t][k],
                                  gsems[t]).wait()

    def compute_chunk(line, half, t):
        bufs = buf_sets[t]
        _, wts = _geometry(loi_v, lams, line, half)

        @plsc.parallel_loop(0, L, unroll=2)
        def pt_body(q):
            if True:
                p = L * half + q
                wb = [None] * 4
                for k in range(4):
                    ws = _splat(wts[k], q)
                    wb[k] = plsc.pack(ws, ws,
                                      format=plsc.PackFormat.INTERLEAVED)
                ie = iota66 + p
                io = ie + OPAD
                for g in range(NG):
                    sg = pl.ds(L * g, L)
                    pe = [None] * 4
                    po = [None] * 4
                    for k in range(4):
                        cw = plsc.bitcast(bufs[k][q, sg], jnp.bfloat16)
                        prod = cw * wb[k]
                        pe[k], po[k] = plsc.unpack(
                            prod, format=plsc.PackFormat.INTERLEAVED)
                    acc_e = (pe[0] + pe[1]) + (pe[2] + pe[3])
                    acc_o = (po[0] + po[1]) + (po[2] + po[3])
                    og = out_t.at[pl.ds(32 * OPAD * g, 32 * OPAD)]
                    plsc.store_scatter(og, [ie], acc_e)
                    plsc.store_scatter(og, [io], acc_o)

    def compact_and_ship(line, j):
        @pl.when(j > 0)
        def _():
            pltpu.make_async_copy(cbuf, out.at[base], osem).wait()

        @plsc.parallel_loop(0, C // 16, unroll=2)
        def row_body(k):
            sb = k * (16 * OPAD)
            kk = k * (16 * N_PTS)
            for r in range(16):
                a = out_t[pl.ds(sb + OPAD * r, L)]
                b = out_t[pl.ds(sb + OPAD * r + L, L)]
                cbuf[pl.ds(kk + N_PTS * r, L)] = a
                cbuf[pl.ds(kk + N_PTS * r + L, L)] = b
        pltpu.async_copy(cbuf, out.at[base + line], osem)

    fire(0, 0, 0)
    fire(0, 1, 1)

    last = LINES_PER_W - 1

    def tri_body(jj, carry):
        lbase = 3 * jj
        for u in range(6):
            line = jnp.minimum(lbase + (u >> 1), last)
            fire(jnp.minimum(lbase + ((u + 2) >> 1), last), (u + 2) & 1,
                 (u + 2) % 3)
            drain_gathers(u % 3)
            compute_chunk(line, u & 1, u % 3)
            if u & 1:
                compact_and_ship(line, 3 * jj + (u >> 1))
        return carry

    lax.fori_loop(0, (LINES_PER_W // 3) + 1, tri_body, 0)

    drain_gathers(0)
    drain_gathers(1)
    pltpu.make_async_copy(cbuf, out.at[base], osem).wait()


_loi_call = functools.partial(
    pl.kernel,
    mesh=plsc.VectorSubcoreMesh(core_axis_name="c", subcore_axis_name="s"),
    out_type=jax.ShapeDtypeStruct((R, C * N_PTS), jnp.float32),
    compiler_params=pltpu.CompilerParams(needs_layout_passes=False),
    scratch_types=(
        [pltpu.VMEM((3, N_PTS), jnp.float32),
         pltpu.VMEM((LINES_PER_W * 6 + 16,), jnp.float32)]
        + [pltpu.VMEM((L,), jnp.int32) for _ in range(12)]
        + [pltpu.VMEM((L, C // 2), jnp.float32) for _ in range(12)]
        + [pltpu.VMEM((C * OPAD,), jnp.float32),
           pltpu.VMEM((C * N_PTS,), jnp.float32)]
        + [pltpu.SemaphoreType.DMA for _ in range(4)]
    ),
)(_loi_body)


def kernel(feature, loi_pred, lambda_):
    table = _make_table(feature)
    loi = loi_pred.reshape(R * 6)
    lam = jnp.transpose(lambda_)
    out = _loi_call(table, loi, lam)
    return out.reshape(R, C, N_PTS)


# speedup vs baseline: 1.0130x; 1.0130x over previous
import functools

import jax
import jax.numpy as jnp
from jax import lax
from jax.experimental import pallas as pl
from jax.experimental.pallas import tpu as pltpu
from jax.experimental.pallas import tpu_sc as plsc

C, H, W = 768, 128, 128
R = 4096
N_PTS = 32
NC, NS, L = 2, 16, 16
NW = NC * NS
LINES_PER_W = R // NW
NG = C // 32
OPAD = N_PTS + 1

_GDN = lax.GatherDimensionNumbers(
    offset_dims=(), collapsed_slice_dims=(0,), start_index_map=(0,))


def _splat(vec, p):
    idx = jnp.full((L, 1), p, dtype=jnp.int32)
    return lax.gather(vec, idx, _GDN, (1,),
                      mode=lax.GatherScatterMode.PROMISE_IN_BOUNDS)


def _rne_bf16_bits(x):
    u = lax.bitcast_convert_type(x, jnp.int32)
    return (u + 0x7FFF + ((u >> 16) & 1)) >> 16


def _transpose_body(x_ref, o_ref):
    x = x_ref[...].reshape(C // 2, 2, -1)
    we = _rne_bf16_bits(x[:, 0, :]) & 0xFFFF
    wo = _rne_bf16_bits(x[:, 1, :]) << 16
    packed = jnp.transpose(we | wo, (1, 0))
    o_ref[...] = lax.bitcast_convert_type(packed, jnp.float32)


def _make_table(feature):
    return pl.pallas_call(
        _transpose_body,
        grid=(16,),
        in_specs=[pl.BlockSpec((C, H * W // 16), lambda i: (0, i))],
        out_specs=pl.BlockSpec((H * W // 16, C // 2), lambda i: (i, 0)),
        out_shape=jax.ShapeDtypeStruct((H * W, C // 2), jnp.float32),
    )(feature.reshape(C, H * W))


def _geometry(loi_v, lams, line, half):
    ctrl = loi_v[pl.ds(line * 6, L)]
    x0 = _splat(ctrl, 0)
    y0 = _splat(ctrl, 1)
    x1 = _splat(ctrl, 2)
    y1 = _splat(ctrl, 3)
    x2 = _splat(ctrl, 4)
    y2 = _splat(ctrl, 5)
    l0, l1, l2 = lams[half]
    px = l0 * x0 + l1 * x1 + l2 * x2 - 0.5
    py = l0 * y0 + l1 * y1 + l2 * y2 - 0.5
    tx = px.astype(jnp.int32).astype(jnp.float32)
    fx = jnp.where(tx > px, tx - 1.0, tx)
    ty = py.astype(jnp.int32).astype(jnp.float32)
    fy = jnp.where(ty > py, ty - 1.0, ty)
    px0 = jnp.minimum(jnp.maximum(fx, 0.0), W - 1.0)
    py0 = jnp.minimum(jnp.maximum(fy, 0.0), H - 1.0)
    px1 = jnp.minimum(px0 + 1.0, W - 1.0)
    py1 = jnp.minimum(py0 + 1.0, H - 1.0)
    wxa = px1 - px
    wxb = px - px0
    wya = py1 - py
    wyb = py - py0
    ix0 = px0.astype(jnp.int32)
    ix1 = px1.astype(jnp.int32)
    iy0 = py0.astype(jnp.int32) * W
    iy1 = py1.astype(jnp.int32) * W
    idx = (iy0 + ix0, iy1 + ix0, iy0 + ix1, iy1 + ix1)
    wts = (wya * wxa, wyb * wxa, wya * wxb, wyb * wxb)
    return idx, wts


def _loi_body(table, loi, lam, out, lam_v, loi_v,
              ia0, ib0, ic0, id0, ia1, ib1, ic1, id1,
              ba0, bb0, bc0, bd0, ba1, bb1, bc1, bd1,
              out_t, cbuf, gsem0, gsem1, osem):
    wid = lax.axis_index("s") * NC + lax.axis_index("c")
    base = wid * LINES_PER_W
    pltpu.sync_copy(lam, lam_v)
    pltpu.sync_copy(loi.at[pl.ds(base * 6, LINES_PER_W * 6)],
                    loi_v.at[pl.ds(0, LINES_PER_W * 6)])
    iota = lax.iota(jnp.int32, L)
    iota66 = iota * (2 * OPAD)
    lams = tuple(
        (lam_v[0, pl.ds(L * h, L)], lam_v[1, pl.ds(L * h, L)],
         lam_v[2, pl.ds(L * h, L)])
        for h in range(2))
    idx_sets = ((ia0, ib0, ic0, id0), (ia1, ib1, ic1, id1))
    buf_sets = ((ba0, bb0, bc0, bd0), (ba1, bb1, bc1, bd1))
    gsems = (gsem0, gsem1)

    def fire(line, half, t):
        idx, _ = _geometry(loi_v, lams, line, half)
        for k in range(4):
            idx_sets[t][k][...] = idx[k]
        for k in range(4):
            pltpu.async_copy(table.at[idx_sets[t][k]], buf_sets[t][k],
                             gsems[t])

    def drain_gathers(t):
        for k in range(4):
            pltpu.make_async_copy(table.at[idx_sets[t][k]], buf_sets[t][k],
                                  gsems[t]).wait()

    def compute_chunk(line, half, t):
        bufs = buf_sets[t]
        _, wts = _geometry(loi_v, lams, line, half)

        @plsc.parallel_loop(0, L, unroll=2)
        def pt_body(q):
            if True:
                p = L * half + q
                wb = [None] * 4
                for k in range(4):
                    ws = _splat(wts[k], q)
                    wb[k] = plsc.pack(ws, ws,
                                      format=plsc.PackFormat.INTERLEAVED)
                ie = iota66 + p
                io = ie + OPAD
                for g in range(NG):
                    sg = pl.ds(L * g, L)
                    pe = [None] * 4
                    po = [None] * 4
                    for k in range(4):
                        cw = plsc.bitcast(bufs[k][q, sg], jnp.bfloat16)
                        prod = cw * wb[k]
                        pe[k], po[k] = plsc.unpack(
                            prod, format=plsc.PackFormat.INTERLEAVED)
                    acc_e = (pe[0] + pe[1]) + (pe[2] + pe[3])
                    acc_o = (po[0] + po[1]) + (po[2] + po[3])
                    og = out_t.at[pl.ds(32 * OPAD * g, 32 * OPAD)]
                    plsc.store_scatter(og, [ie], acc_e)
                    plsc.store_scatter(og, [io], acc_o)

    def compact_and_ship(line, j):
        @pl.when(j > 0)
        def _():
            pltpu.make_async_copy(cbuf, out.at[base], osem).wait()

        @plsc.parallel_loop(0, C // 16, unroll=2)
        def row_body(k):
            sb = k * (16 * OPAD)
            kk = k * (16 * N_PTS)
            for r in range(16):
                a = out_t[pl.ds(sb + OPAD * r, L)]
                b = out_t[pl.ds(sb + OPAD * r + L, L)]
                cbuf[pl.ds(kk + N_PTS * r, L)] = a
                cbuf[pl.ds(kk + N_PTS * r + L, L)] = b
        pltpu.async_copy(cbuf, out.at[base + line], osem)

    fire(0, 0, 0)

    def line_body(j, carry):
        for half in range(2):
            t = half
            if half == 0:
                fire(j, 1, 1)
            else:
                fire(jnp.minimum(j + 1, LINES_PER_W - 1), 0, 0)
            drain_gathers(t)
            compute_chunk(j, half, t)
        compact_and_ship(j, j)
        return carry

    lax.fori_loop(0, LINES_PER_W, line_body, 0)

    drain_gathers(0)
    pltpu.make_async_copy(cbuf, out.at[base], osem).wait()


_loi_call = functools.partial(
    pl.kernel,
    mesh=plsc.VectorSubcoreMesh(core_axis_name="c", subcore_axis_name="s"),
    out_type=jax.ShapeDtypeStruct((R, C * N_PTS), jnp.float32),
    compiler_params=pltpu.CompilerParams(needs_layout_passes=False),
    scratch_types=(
        [pltpu.VMEM((3, N_PTS), jnp.float32),
         pltpu.VMEM((LINES_PER_W * 6 + 16,), jnp.float32)]
        + [pltpu.VMEM((L,), jnp.int32) for _ in range(8)]
        + [pltpu.VMEM((L, C // 2), jnp.float32) for _ in range(8)]
        + [pltpu.VMEM((C * OPAD,), jnp.float32),
           pltpu.VMEM((C * N_PTS,), jnp.float32)]
        + [pltpu.SemaphoreType.DMA for _ in range(3)]
    ),
)(_loi_body)


def kernel(feature, loi_pred, lambda_):
    table = _make_table(feature)
    loi = loi_pred.reshape(R * 6)
    lam = jnp.transpose(lambda_)
    out = _loi_call(table, loi, lam)
    return out.reshape(R, C, N_PTS)
